# baseline (speedup 1.0000x reference)
import jax
import jax.numpy as jnp
import numpy as np
from jax.experimental import pallas as pl
from jax.experimental.pallas import tpu as pltpu

_TEMP = 0.8
_K = 50
_B = 128
_V = 100000
_W = 2048
_NF = 48
_TOFF = _NF * _W
_TW = _V - _TOFF
_BR = 8
_NEG = np.float32(-1e30)
_INT_MIN = np.int32(-2147483648)


def _monotonic(bits):
    return jnp.where(bits < 0, bits ^ jnp.int32(0x7FFFFFFF), bits)


def _rotl(x, d):
    return jnp.left_shift(x, d) | jax.lax.shift_right_logical(x, 32 - d)


def _threefry_bits(p):
    ks0 = jnp.int32(0)
    ks1 = jnp.int32(1234)
    ks2 = jnp.int32(1234 ^ 0x1BD11BDA)
    x0 = jnp.zeros_like(p)
    x1 = p + ks1
    r0 = (13, 15, 26, 6)
    r1 = (17, 29, 16, 24)
    sched = ((r0, ks1, ks2, 1), (r1, ks2, ks0, 2), (r0, ks0, ks1, 3),
             (r1, ks1, ks2, 4), (r0, ks2, ks0, 5))
    for rs, a, b, i in sched:
        for r in rs:
            x0 = x0 + x1
            x1 = _rotl(x1, r)
            x1 = x0 ^ x1
        x0 = x0 + a
        x1 = x1 + b + jnp.int32(i)
    return x0 ^ x1


def _gumbel_from_bits(bits):
    fb = jax.lax.shift_right_logical(bits, 9) | jnp.int32(0x3F800000)
    u = jax.lax.bitcast_convert_type(fb, jnp.float32) - jnp.float32(1.0)
    tiny = jnp.float32(1.1754943508222875e-38)
    uu = jnp.maximum(tiny, u * (jnp.float32(1.0) - tiny) + tiny)
    return -jnp.log(-jnp.log(uu))


def _body(x_ref, probs_ref, nt_ref, yh_ref, yl_ref, best_ref, bidx_ref):
    blk = pl.program_id(0)
    one16 = np.int16(1)
    zero16 = np.int16(0)

    def p1_piece(off, w, m):
        sl = pl.ds(off, w)
        l = x_ref[:, sl] / jnp.float32(_TEMP)
        u = _monotonic(jax.lax.bitcast_convert_type(l, jnp.int32)) ^ _INT_MIN
        h = jax.lax.shift_right_logical(u, 16)
        lo = u & jnp.int32(0xFFFF)
        yh_ref[:, sl] = (h - jnp.int32(32768)).astype(jnp.int16)
        yl_ref[:, sl] = (lo - jnp.int32(32768)).astype(jnp.int16)
        return jnp.maximum(m, jnp.max(l, axis=1, keepdims=True))

    m = jax.lax.fori_loop(
        0, _NF,
        lambda ch, m: p1_piece(pl.multiple_of(ch * _W, _W), _W, m),
        jnp.full((_BR, 1), -jnp.inf, jnp.float32))
    m = p1_piece(_TOFF, _TW, m)

    def _count(pred_fn):
        def pcnt(ch, acc):
            off = pl.multiple_of(ch * _W, _W)
            return acc + jnp.where(pred_fn(off, _W), one16, zero16)

        acc = jax.lax.fori_loop(0, _NF, pcnt,
                                jnp.zeros((_BR, _W), jnp.int16))
        cnt = jnp.sum(acc.astype(jnp.int32), axis=1, keepdims=True)
        tail = jnp.sum(pred_fn(_TOFF, _TW).astype(jnp.int32), axis=1,
                       keepdims=True)
        return cnt + tail

    def _cnt_hi(s16):
        return _count(lambda off, w: yh_ref[:, pl.ds(off, w)] >= s16)

    def pbit_hi(t, pref):
        bit = 15 - t
        cand = pref | jnp.left_shift(jnp.int32(1), bit)
        cnt = _cnt_hi((cand - jnp.int32(32768)).astype(jnp.int16))
        return jnp.where(cnt >= _K, cand, pref)

    pref_h = jax.lax.fori_loop(0, 16, pbit_hi,
                               jnp.zeros((_BR, 1), jnp.int32))
    ph16 = (pref_h - jnp.int32(32768)).astype(jnp.int16)

    cnt_gt = _cnt_hi((pref_h + 1 - jnp.int32(32768)).astype(jnp.int16))
    cnt_gt = jnp.where(pref_h >= jnp.int32(65535),
                       jnp.zeros_like(cnt_gt), cnt_gt)

    def pbit_lo(t, pref):
        bit = 15 - t
        cand = pref | jnp.left_shift(jnp.int32(1), bit)
        s_l16 = (cand - jnp.int32(32768)).astype(jnp.int16)
        cnt = cnt_gt + _count(
            lambda off, w: (yh_ref[:, pl.ds(off, w)] == ph16)
            & (yl_ref[:, pl.ds(off, w)] >= s_l16))
        return jnp.where(cnt >= _K, cand, pref)

    pref_l = jax.lax.fori_loop(0, 16, pbit_lo,
                               jnp.zeros((_BR, 1), jnp.int32))

    u_star = jnp.left_shift(pref_h, 16) | pref_l
    y_star = u_star ^ _INT_MIN
    bits_star = jnp.where(y_star < 0, y_star ^ jnp.int32(0x7FFFFFFF), y_star)
    t_f = jax.lax.bitcast_convert_type(bits_star, jnp.float32)

    row_v = (blk * _BR
             + jax.lax.broadcasted_iota(jnp.int32, (_BR, 1), 0)) * _V
    best_ref[...] = jnp.full((_BR, 128), -jnp.inf, jnp.float32)
    bidx_ref[...] = jnp.full((_BR, 128), 2**30, jnp.int32)

    def p3_piece(off, w, sacc):
        sl = pl.ds(off, w)
        l = x_ref[:, sl] / jnp.float32(_TEMP)
        kept = l >= t_f
        e = jnp.where(kept, jnp.exp(l - m), jnp.float32(0.0))
        probs_ref[:, sl] = e
        sacc = sacc + jnp.sum(e, axis=1, keepdims=True)
        for s0 in range(0, w, 128):
            sw = min(128, w - s0)
            kept_sl = jax.lax.slice_in_dim(kept, s0, s0 + sw, axis=1)

            @pl.when(jnp.any(kept_sl))
            def _():
                l_sl = jax.lax.slice_in_dim(l, s0, s0 + sw, axis=1)
                vg = off + s0 + jax.lax.broadcasted_iota(
                    jnp.int32, (_BR, sw), 1)
                g = _gumbel_from_bits(_threefry_bits(row_v + vg))
                score = jnp.where(kept_sl, l_sl + g, _NEG)
                cb = jnp.max(score, axis=1, keepdims=True)
                ci = jnp.min(jnp.where(score == cb, vg, jnp.int32(2**30)),
                             axis=1, keepdims=True)
                best = best_ref[:, 0:1]
                bidx = bidx_ref[:, 0:1]
                upd = (cb > best) | ((cb == best) & (ci < bidx))
                best_ref[...] = jnp.broadcast_to(
                    jnp.where(upd, cb, best), (_BR, 128))
                bidx_ref[...] = jnp.broadcast_to(
                    jnp.where(upd, ci, bidx), (_BR, 128))

        return sacc

    sacc = jax.lax.fori_loop(
        0, _NF,
        lambda ch, sacc: p3_piece(pl.multiple_of(ch * _W, _W), _W, sacc),
        jnp.zeros((_BR, 1), jnp.float32))
    sacc = p3_piece(_TOFF, _TW, sacc)

    inv = jnp.float32(1.0) / sacc

    def p4_piece(off, w):
        sl = pl.ds(off, w)
        probs_ref[:, sl] = probs_ref[:, sl] * inv

    def p4(ch, c):
        p4_piece(pl.multiple_of(ch * _W, _W), _W)
        return c

    jax.lax.fori_loop(0, _NF, p4, 0)
    p4_piece(_TOFF, _TW)
    nt_ref[...] = bidx_ref[...]


def kernel(x):
    probs, nt = pl.pallas_call(
        _body,
        grid=(_B // _BR,),
        in_specs=[pl.BlockSpec((_BR, _V), lambda i: (i, 0))],
        out_specs=[pl.BlockSpec((_BR, _V), lambda i: (i, 0)),
                   pl.BlockSpec((_BR, 128), lambda i: (i, 0))],
        out_shape=[jax.ShapeDtypeStruct((_B, _V), jnp.float32),
                   jax.ShapeDtypeStruct((_B, 128), jnp.int32)],
        scratch_shapes=[pltpu.VMEM((_BR, _V), jnp.int16),
                        pltpu.VMEM((_BR, _V), jnp.int16),
                        pltpu.VMEM((_BR, 128), jnp.float32),
                        pltpu.VMEM((_BR, 128), jnp.int32)],
    )(x)
    return probs, nt[:, 0]


# speedup vs baseline: 4.3663x; 4.3663x over previous
import jax
import jax.numpy as jnp
import numpy as np
from jax.experimental import pallas as pl
from jax.experimental.pallas import tpu as pltpu

_TEMP = 0.8
_K = 50
_B = 128
_V = 100000
_W = 2048
_NF = 48
_TOFF = _NF * _W
_TW = _V - _TOFF
_BR = 8
_NEG = np.float32(-1e30)
_INT_MIN = np.int32(-2147483648)


def _monotonic(bits):
    return jnp.where(bits < 0, bits ^ jnp.int32(0x7FFFFFFF), bits)


def _rotl(x, d):
    return jnp.left_shift(x, d) | jax.lax.shift_right_logical(x, 32 - d)


def _threefry_bits(p):
    ks0 = jnp.int32(0)
    ks1 = jnp.int32(1234)
    ks2 = jnp.int32(1234 ^ 0x1BD11BDA)
    x0 = jnp.zeros_like(p)
    x1 = p + ks1
    r0 = (13, 15, 26, 6)
    r1 = (17, 29, 16, 24)
    sched = ((r0, ks1, ks2, 1), (r1, ks2, ks0, 2), (r0, ks0, ks1, 3),
             (r1, ks1, ks2, 4), (r0, ks2, ks0, 5))
    for rs, a, b, i in sched:
        for r in rs:
            x0 = x0 + x1
            x1 = _rotl(x1, r)
            x1 = x0 ^ x1
        x0 = x0 + a
        x1 = x1 + b + jnp.int32(i)
    return x0 ^ x1


def _gumbel_from_bits(bits):
    fb = jax.lax.shift_right_logical(bits, 9) | jnp.int32(0x3F800000)
    u = jax.lax.bitcast_convert_type(fb, jnp.float32) - jnp.float32(1.0)
    tiny = jnp.float32(1.1754943508222875e-38)
    uu = jnp.maximum(tiny, u * (jnp.float32(1.0) - tiny) + tiny)
    return -jnp.log(-jnp.log(uu))


def _body(x_ref, probs_ref, nt_ref, yh_ref, yl_ref):
    blk = pl.program_id(0)
    one16 = np.int16(1)
    zero16 = np.int16(0)

    def p1_piece(off, w, m):
        sl = pl.ds(off, w)
        l = x_ref[:, sl] / jnp.float32(_TEMP)
        u = _monotonic(jax.lax.bitcast_convert_type(l, jnp.int32)) ^ _INT_MIN
        h = jax.lax.shift_right_logical(u, 16)
        lo = u & jnp.int32(0xFFFF)
        yh_ref[:, sl] = (h - jnp.int32(32768)).astype(jnp.int16)
        yl_ref[:, sl] = (lo - jnp.int32(32768)).astype(jnp.int16)
        return jnp.maximum(m, jnp.max(l, axis=1, keepdims=True))

    m = jax.lax.fori_loop(
        0, _NF,
        lambda ch, m: p1_piece(pl.multiple_of(ch * _W, _W), _W, m),
        jnp.full((_BR, 1), -jnp.inf, jnp.float32))
    m = p1_piece(_TOFF, _TW, m)

    def _count(pred_fn):
        def pcnt(ch, acc):
            off = pl.multiple_of(ch * _W, _W)
            return acc + jnp.where(pred_fn(off, _W), one16, zero16)

        acc = jax.lax.fori_loop(0, _NF, pcnt,
                                jnp.zeros((_BR, _W), jnp.int16))
        cnt = jnp.sum(acc.astype(jnp.int32), axis=1, keepdims=True)
        tail = jnp.sum(pred_fn(_TOFF, _TW).astype(jnp.int32), axis=1,
                       keepdims=True)
        return cnt + tail

    def _cnt_hi(s16):
        return _count(lambda off, w: yh_ref[:, pl.ds(off, w)] >= s16)

    def pbit_hi(t, pref):
        bit = 15 - t
        cand = pref | jnp.left_shift(jnp.int32(1), bit)
        cnt = _cnt_hi((cand - jnp.int32(32768)).astype(jnp.int16))
        return jnp.where(cnt >= _K, cand, pref)

    pref_h = jax.lax.fori_loop(0, 16, pbit_hi,
                               jnp.zeros((_BR, 1), jnp.int32))
    ph16 = (pref_h - jnp.int32(32768)).astype(jnp.int16)

    cnt_gt = _cnt_hi((pref_h + 1 - jnp.int32(32768)).astype(jnp.int16))
    cnt_gt = jnp.where(pref_h >= jnp.int32(65535),
                       jnp.zeros_like(cnt_gt), cnt_gt)

    def pbit_lo(t, pref):
        bit = 15 - t
        cand = pref | jnp.left_shift(jnp.int32(1), bit)
        s_l16 = (cand - jnp.int32(32768)).astype(jnp.int16)
        cnt = cnt_gt + _count(
            lambda off, w: (yh_ref[:, pl.ds(off, w)] == ph16)
            & (yl_ref[:, pl.ds(off, w)] >= s_l16))
        return jnp.where(cnt >= _K, cand, pref)

    pref_l = jax.lax.fori_loop(0, 16, pbit_lo,
                               jnp.zeros((_BR, 1), jnp.int32))

    u_star = jnp.left_shift(pref_h, 16) | pref_l
    y_star = u_star ^ _INT_MIN
    bits_star = jnp.where(y_star < 0, y_star ^ jnp.int32(0x7FFFFFFF), y_star)
    t_f = jax.lax.bitcast_convert_type(bits_star, jnp.float32)

    row_v = (blk * _BR
             + jax.lax.broadcasted_iota(jnp.int32, (_BR, 1), 0)) * _V

    def p3_piece(off, w, carry):
        sacc, best, bidx = carry
        sl = pl.ds(off, w)
        l = x_ref[:, sl] / jnp.float32(_TEMP)
        kept = l >= t_f
        e = jnp.where(kept, jnp.exp(l - m), jnp.float32(0.0))
        probs_ref[:, sl] = e
        sacc = sacc + jnp.sum(e, axis=1, keepdims=True)
        vg = off + jax.lax.broadcasted_iota(jnp.int32, (_BR, w), 1)
        g = _gumbel_from_bits(_threefry_bits(row_v + vg))
        score = jnp.where(kept, l + g, _NEG)
        cb = jnp.max(score, axis=1, keepdims=True)
        ci = jnp.min(jnp.where(score == cb, vg, jnp.int32(2**30)),
                     axis=1, keepdims=True)
        upd = (cb > best) | ((cb == best) & (ci < bidx))
        best = jnp.where(upd, cb, best)
        bidx = jnp.where(upd, ci, bidx)
        return sacc, best, bidx

    sacc, _, bidx = p3_piece(_TOFF, _TW, jax.lax.fori_loop(
        0, _NF,
        lambda ch, c: p3_piece(pl.multiple_of(ch * _W, _W), _W, c),
        (jnp.zeros((_BR, 1), jnp.float32),
         jnp.full((_BR, 1), -jnp.inf, jnp.float32),
         jnp.full((_BR, 1), 2**30, jnp.int32))))

    inv = jnp.float32(1.0) / sacc

    def p4_piece(off, w):
        sl = pl.ds(off, w)
        probs_ref[:, sl] = probs_ref[:, sl] * inv

    def p4(ch, c):
        p4_piece(pl.multiple_of(ch * _W, _W), _W)
        return c

    jax.lax.fori_loop(0, _NF, p4, 0)
    p4_piece(_TOFF, _TW)
    nt_ref[...] = jnp.broadcast_to(bidx, (_BR, 128))


def kernel(x):
    probs, nt = pl.pallas_call(
        _body,
        grid=(_B // _BR,),
        in_specs=[pl.BlockSpec((_BR, _V), lambda i: (i, 0))],
        out_specs=[pl.BlockSpec((_BR, _V), lambda i: (i, 0)),
                   pl.BlockSpec((_BR, 128), lambda i: (i, 0))],
        out_shape=[jax.ShapeDtypeStruct((_B, _V), jnp.float32),
                   jax.ShapeDtypeStruct((_B, 128), jnp.int32)],
        scratch_shapes=[pltpu.VMEM((_BR, _V), jnp.int16),
                        pltpu.VMEM((_BR, _V), jnp.int16)],
    )(x)
    return probs, nt[:, 0]


# speedup vs baseline: 4.5432x; 1.0405x over previous
import jax
import jax.numpy as jnp
import numpy as np
from jax.experimental import pallas as pl
from jax.experimental.pallas import tpu as pltpu

_TEMP = 0.8
_K = 50
_B = 128
_V = 100000
_W = 2048
_NF = 48
_TOFF = _NF * _W
_TW = _V - _TOFF
_BR = 8
_NEG = np.float32(-1e30)
_INT_MIN = np.int32(-2147483648)


def _monotonic(bits):
    return jnp.where(bits < 0, bits ^ jnp.int32(0x7FFFFFFF), bits)


def _rotl(x, d):
    return jnp.left_shift(x, d) | jax.lax.shift_right_logical(x, 32 - d)


def _threefry_bits(p):
    ks0 = jnp.int32(0)
    ks1 = jnp.int32(1234)
    ks2 = jnp.int32(1234 ^ 0x1BD11BDA)
    x0 = jnp.zeros_like(p)
    x1 = p + ks1
    r0 = (13, 15, 26, 6)
    r1 = (17, 29, 16, 24)
    sched = ((r0, ks1, ks2, 1), (r1, ks2, ks0, 2), (r0, ks0, ks1, 3),
             (r1, ks1, ks2, 4), (r0, ks2, ks0, 5))
    for rs, a, b, i in sched:
        for r in rs:
            x0 = x0 + x1
            x1 = _rotl(x1, r)
            x1 = x0 ^ x1
        x0 = x0 + a
        x1 = x1 + b + jnp.int32(i)
    return x0 ^ x1


def _gumbel_from_bits(bits):
    fb = jax.lax.shift_right_logical(bits, 9) | jnp.int32(0x3F800000)
    u = jax.lax.bitcast_convert_type(fb, jnp.float32) - jnp.float32(1.0)
    tiny = jnp.float32(1.1754943508222875e-38)
    uu = jnp.maximum(tiny, u * (jnp.float32(1.0) - tiny) + tiny)
    return -jnp.log(-jnp.log(uu))


def _body(x_ref, probs_ref, nt_ref, yh_ref, yl_ref):
    blk = pl.program_id(0)
    one16 = np.int16(1)
    zero16 = np.int16(0)

    def p1_piece(off, w, m):
        sl = pl.ds(off, w)
        l = x_ref[:, sl] / jnp.float32(_TEMP)
        u = _monotonic(jax.lax.bitcast_convert_type(l, jnp.int32)) ^ _INT_MIN
        h = jax.lax.shift_right_logical(u, 16)
        lo = u & jnp.int32(0xFFFF)
        yh_ref[:, sl] = (h - jnp.int32(32768)).astype(jnp.int16)
        yl_ref[:, sl] = (lo - jnp.int32(32768)).astype(jnp.int16)
        return jnp.maximum(m, jnp.max(l, axis=1, keepdims=True))

    m = jax.lax.fori_loop(
        0, _NF,
        lambda ch, m: p1_piece(pl.multiple_of(ch * _W, _W), _W, m),
        jnp.full((_BR, 1), -jnp.inf, jnp.float32))
    m = p1_piece(_TOFF, _TW, m)

    def _count(pred_fn):
        def pcnt(ch, acc):
            off = pl.multiple_of(ch * _W, _W)
            return acc + jnp.where(pred_fn(off, _W), one16, zero16)

        acc = jax.lax.fori_loop(0, _NF, pcnt,
                                jnp.zeros((_BR, _W), jnp.int16))
        cnt = jnp.sum(acc.astype(jnp.int32), axis=1, keepdims=True)
        tail = jnp.sum(pred_fn(_TOFF, _TW).astype(jnp.int32), axis=1,
                       keepdims=True)
        return cnt + tail

    def _cnt_hi(s16):
        return _count(lambda off, w: yh_ref[:, pl.ds(off, w)] >= s16)

    def pbit_hi(t, pref):
        bit = 15 - t
        cand = pref | jnp.left_shift(jnp.int32(1), bit)
        cnt = _cnt_hi((cand - jnp.int32(32768)).astype(jnp.int16))
        return jnp.where(cnt >= _K, cand, pref)

    pref_h = jax.lax.fori_loop(0, 16, pbit_hi,
                               jnp.zeros((_BR, 1), jnp.int32))
    ph16 = (pref_h - jnp.int32(32768)).astype(jnp.int16)

    cnt_gt = _cnt_hi((pref_h + 1 - jnp.int32(32768)).astype(jnp.int16))
    cnt_gt = jnp.where(pref_h >= jnp.int32(65535),
                       jnp.zeros_like(cnt_gt), cnt_gt)

    def pz(ch, c):
        sl = pl.ds(pl.multiple_of(ch * _W, _W), _W)
        yh_ref[:, sl] = jnp.where(yh_ref[:, sl] == ph16, yl_ref[:, sl],
                                  np.int16(-32768))
        return c

    jax.lax.fori_loop(0, _NF, pz, 0)
    tl = pl.ds(_TOFF, _TW)
    yh_ref[:, tl] = jnp.where(yh_ref[:, tl] == ph16, yl_ref[:, tl],
                              np.int16(-32768))

    def pbit_lo(t, pref):
        bit = 15 - t
        cand = pref | jnp.left_shift(jnp.int32(1), bit)
        s_l16 = (cand - jnp.int32(32768)).astype(jnp.int16)
        cnt = cnt_gt + _count(lambda off, w: yh_ref[:, pl.ds(off, w)] >= s_l16)
        return jnp.where(cnt >= _K, cand, pref)

    pref_l = jax.lax.fori_loop(0, 16, pbit_lo,
                               jnp.zeros((_BR, 1), jnp.int32))

    u_star = jnp.left_shift(pref_h, 16) | pref_l
    y_star = u_star ^ _INT_MIN
    bits_star = jnp.where(y_star < 0, y_star ^ jnp.int32(0x7FFFFFFF), y_star)
    t_f = jax.lax.bitcast_convert_type(bits_star, jnp.float32)

    row_v = (blk * _BR
             + jax.lax.broadcasted_iota(jnp.int32, (_BR, 1), 0)) * _V

    def p3_piece(off, w, carry):
        sacc, best, bidx = carry
        sl = pl.ds(off, w)
        l = x_ref[:, sl] / jnp.float32(_TEMP)
        kept = l >= t_f
        e = jnp.where(kept, jnp.exp(l - m), jnp.float32(0.0))
        probs_ref[:, sl] = e
        sacc = sacc + jnp.sum(e, axis=1, keepdims=True)
        vg = off + jax.lax.broadcasted_iota(jnp.int32, (_BR, w), 1)
        g = _gumbel_from_bits(_threefry_bits(row_v + vg))
        score = jnp.where(kept, l + g, _NEG)
        cb = jnp.max(score, axis=1, keepdims=True)
        ci = jnp.min(jnp.where(score == cb, vg, jnp.int32(2**30)),
                     axis=1, keepdims=True)
        upd = (cb > best) | ((cb == best) & (ci < bidx))
        best = jnp.where(upd, cb, best)
        bidx = jnp.where(upd, ci, bidx)
        return sacc, best, bidx

    sacc, _, bidx = p3_piece(_TOFF, _TW, jax.lax.fori_loop(
        0, _NF,
        lambda ch, c: p3_piece(pl.multiple_of(ch * _W, _W), _W, c),
        (jnp.zeros((_BR, 1), jnp.float32),
         jnp.full((_BR, 1), -jnp.inf, jnp.float32),
         jnp.full((_BR, 1), 2**30, jnp.int32))))

    inv = jnp.float32(1.0) / sacc

    def p4_piece(off, w):
        sl = pl.ds(off, w)
        probs_ref[:, sl] = probs_ref[:, sl] * inv

    def p4(ch, c):
        p4_piece(pl.multiple_of(ch * _W, _W), _W)
        return c

    jax.lax.fori_loop(0, _NF, p4, 0)
    p4_piece(_TOFF, _TW)
    nt_ref[...] = jnp.broadcast_to(bidx, (_BR, 128))


def kernel(x):
    probs, nt = pl.pallas_call(
        _body,
        grid=(_B // _BR,),
        in_specs=[pl.BlockSpec((_BR, _V), lambda i: (i, 0))],
        out_specs=[pl.BlockSpec((_BR, _V), lambda i: (i, 0)),
                   pl.BlockSpec((_BR, 128), lambda i: (i, 0))],
        out_shape=[jax.ShapeDtypeStruct((_B, _V), jnp.float32),
                   jax.ShapeDtypeStruct((_B, 128), jnp.int32)],
        scratch_shapes=[pltpu.VMEM((_BR, _V), jnp.int16),
                        pltpu.VMEM((_BR, _V), jnp.int16)],
    )(x)
    return probs, nt[:, 0]
